# Initial kernel scaffold; baseline (speedup 1.0000x reference)
import jax
import jax.numpy as jnp
from jax.experimental import pallas as pl


def kernel(X, pos_embedding):
    raise NotImplementedError("write your pallas kernel here")


# speedup vs baseline: 2.8757x; 2.8757x over previous
import jax
import jax.numpy as jnp
from jax.experimental import pallas as pl


def _add_kernel(x_ref, pos_ref, out_ref):
    out_ref[...] = x_ref[...] + pos_ref[...][None]


def kernel(X, pos_embedding):
    B, L, D = X.shape
    Lb = 512
    grid = (L // Lb, B)
    return pl.pallas_call(
        _add_kernel,
        grid=grid,
        in_specs=[
            pl.BlockSpec((1, Lb, D), lambda l, b: (b, l, 0)),
            pl.BlockSpec((Lb, D), lambda l, b: (l, 0)),
        ],
        out_specs=pl.BlockSpec((1, Lb, D), lambda l, b: (b, l, 0)),
        out_shape=jax.ShapeDtypeStruct((B, L, D), X.dtype),
    )(X, pos_embedding)


# speedup vs baseline: 3.1396x; 1.0918x over previous
import jax
import jax.numpy as jnp
from jax.experimental import pallas as pl


def _add_kernel(x_ref, pos_ref, out_ref):
    out_ref[...] = x_ref[...] + pos_ref[...][None]


def kernel(X, pos_embedding):
    B, L, D = X.shape
    Lb = 1024
    grid = (L // Lb, B)
    return pl.pallas_call(
        _add_kernel,
        grid=grid,
        in_specs=[
            pl.BlockSpec((1, Lb, D), lambda l, b: (b, l, 0)),
            pl.BlockSpec((Lb, D), lambda l, b: (l, 0)),
        ],
        out_specs=pl.BlockSpec((1, Lb, D), lambda l, b: (b, l, 0)),
        out_shape=jax.ShapeDtypeStruct((B, L, D), X.dtype),
    )(X, pos_embedding)


# speedup vs baseline: 3.3994x; 1.0827x over previous
import jax
import jax.numpy as jnp
from jax.experimental import pallas as pl


def _add_kernel(x_ref, pos_ref, out_ref):
    out_ref[...] = x_ref[...] + pos_ref[...][None]


def kernel(X, pos_embedding):
    B, L, D = X.shape
    Lb = 2048
    grid = (L // Lb, B)
    return pl.pallas_call(
        _add_kernel,
        grid=grid,
        in_specs=[
            pl.BlockSpec((1, Lb, D), lambda l, b: (b, l, 0)),
            pl.BlockSpec((Lb, D), lambda l, b: (l, 0)),
        ],
        out_specs=pl.BlockSpec((1, Lb, D), lambda l, b: (b, l, 0)),
        out_shape=jax.ShapeDtypeStruct((B, L, D), X.dtype),
    )(X, pos_embedding)
